# baseline (speedup 1.0000x reference)
import functools

import jax
import jax.numpy as jnp
from jax import lax
from jax.experimental import pallas as pl
from jax.experimental.pallas import tpu as pltpu
from jax.experimental.pallas import tpu_sc as plsc

NDIM = 16
NKNOT = 32
LANES = 16
NWORKERS = 32

_NC = 10
_T_XX = 0
_T_YY = NDIM * NKNOT
_T_D = 2 * NDIM * NKNOT
_T_C = 3 * NDIM * NKNOT
_T_LEN = 3 * NDIM * NKNOT + NDIM * _NC * LANES

_NCOLS = 1000000
_CHUNK = 512
_SC_K = 42
_SC_COLS = NWORKERS * _SC_K * _CHUNK
_TC_COLS = _NCOLS - _SC_COLS
_BC = 2048

_LN2 = 0.6931471805599453
_LOG_COEF = (
    9.09649109370747e-08,
    0.9999914499316742,
    -0.49980110932854144,
    0.33133371651864063,
    -0.2391898880613915,
    0.16478215592875825,
    -0.09231255283440946,
    0.03441802524375755,
    -0.006074773332369956,
)


def _softlog(a):
    i = lax.bitcast_convert_type(a, jnp.int32)
    e = (i >> 23) - 127
    z = lax.bitcast_convert_type(
        (i & 0x007FFFFF) | 0x3F800000, jnp.float32) - 1.0
    p = jnp.full((LANES,), _LOG_COEF[8], jnp.float32)
    for k in range(7, -1, -1):
        p = p * z + _LOG_COEF[k]
    return e.astype(jnp.float32) * _LN2 + p


def _spline_vec(tab_v, xv, bvec, b16, consts):
    xx0, xx31, d0, cleft, d31, cright, ld0, ld31, xmid = consts
    c = jnp.where(xmid < xv, b16, bvec)
    for s in (8, 4, 2, 1):
        v = plsc.load_gather(tab_v, [c + (s - 1)])
        c = jnp.where(v < xv, c + s, c)
    lo = jnp.maximum(c - 1, bvec)
    lo_y = lo + _T_YY
    lo_d = lo + _T_D
    x_lo = plsc.load_gather(tab_v, [lo])
    x_hi = plsc.load_gather(tab_v, [lo + 1])
    y_lo = plsc.load_gather(tab_v, [lo_y])
    y_hi = plsc.load_gather(tab_v, [lo_y + 1])
    d_lo = plsc.load_gather(tab_v, [lo_d])
    d_hi = plsc.load_gather(tab_v, [lo_d + 1])

    rdx = 1.0 / (x_hi - x_lo)
    xi = (xv - x_lo) * rdx
    dy = y_hi - y_lo
    sl = dy * rdx
    u = xi * (1.0 - xi)
    xi1 = 1.0 - xi
    r = (sl + sl) * u
    den = (d_hi + d_lo) * u - r + sl
    rden = 1.0 / den
    xi2 = xi * xi
    y_mid = y_lo + dy * rden * (sl * xi2 + d_lo * u)
    num = d_lo * (xi1 * xi1) + (d_hi * xi2 + r)
    t = sl * rden
    ld_mid = _softlog((t * t) * num)

    y_left = cleft + xv * d0
    y_right = cright + xv * d31
    sel0 = jnp.logical_not(xx0 < xv)
    seln = xx31 < xv
    y = jnp.where(sel0, y_left, jnp.where(seln, y_right, y_mid))
    ld = jnp.where(sel0, ld0, jnp.where(seln, ld31, ld_mid))
    return y, ld


def _do_chunk(tab_v, xb, yb, ldb, ncols):
    per_dim = ncols // LANES
    shift = per_dim.bit_length() - 1
    assert per_dim == 1 << shift

    @plsc.parallel_loop(0, NDIM * per_dim, unroll=16)
    def vec_body(j):
        d = j >> shift
        sl_ = pl.ds((j - (d << shift)) * LANES, LANES)
        cbase = _T_C + d * (_NC * LANES)
        consts = tuple(
            tab_v[pl.ds(cbase + i * LANES, LANES)] for i in range(9))
        bvec = jnp.full((LANES,), d * NKNOT, jnp.int32)
        b16 = jnp.full((LANES,), d * NKNOT + 16, jnp.int32)
        y, ld = _spline_vec(tab_v, xb[d, sl_], bvec, b16, consts)
        yb[d, sl_] = y
        ldb[d, sl_] = ld


def _sc_body(x_hbm, tab_hbm, y_hbm, ld_hbm,
             tab_v, xb, yb, ldb, sin, soy, sold):
    wid = lax.axis_index("s") * 2 + lax.axis_index("c")
    base = wid * (_SC_K * _CHUNK)
    nchunks = _SC_K

    pltpu.sync_copy(tab_hbm, tab_v)

    def start_in(g):
        b = g & 1
        c0 = base + g * _CHUNK
        pltpu.async_copy(x_hbm.at[:, pl.ds(c0, _CHUNK)], xb.at[b],
                         sin.at[b])

    start_in(0)
    start_in(1)

    def chunk_body(g, carry):
        b = g & 1
        c0 = base + g * _CHUNK
        pltpu.make_async_copy(x_hbm.at[:, pl.ds(c0, _CHUNK)], xb.at[b],
                              sin.at[b]).wait()

        @pl.when(g >= 2)
        def _drain():
            c0p = c0 - 2 * _CHUNK
            pltpu.make_async_copy(yb.at[b], y_hbm.at[:, pl.ds(c0p, _CHUNK)],
                                  soy.at[b]).wait()
            pltpu.make_async_copy(ldb.at[b], ld_hbm.at[:, pl.ds(c0p, _CHUNK)],
                                  sold.at[b]).wait()

        _do_chunk(tab_v, xb.at[b], yb.at[b], ldb.at[b], _CHUNK)
        pltpu.async_copy(yb.at[b], y_hbm.at[:, pl.ds(c0, _CHUNK)], soy.at[b])
        pltpu.async_copy(ldb.at[b], ld_hbm.at[:, pl.ds(c0, _CHUNK)],
                         sold.at[b])

        @pl.when(g + 2 < nchunks)
        def _prefetch():
            start_in(g + 2)

        return carry

    lax.fori_loop(0, nchunks, chunk_body, 0)

    for off in (2, 1):
        g = nchunks - off
        b = g & 1
        c0 = base + g * _CHUNK
        pltpu.make_async_copy(yb.at[b], y_hbm.at[:, pl.ds(c0, _CHUNK)],
                              soy.at[b]).wait()
        pltpu.make_async_copy(ldb.at[b], ld_hbm.at[:, pl.ds(c0, _CHUNK)],
                              sold.at[b]).wait()


def _tc_body(xx_r, yy_r, dd_r, lds_r, dxx_r, dyy_r, ddl_r,
             x_r, y_r, ld_r):
    x = x_r[...]
    xx = xx_r[...]
    yy = yy_r[...]
    dd = dd_r[...]
    lds = lds_r[...]
    dxx = dxx_r[...]
    dyy = dyy_r[...]
    ddl = ddl_r[...]
    shp = x.shape

    x_lo = jnp.broadcast_to(xx[:, 0:1], shp)
    x_hi = jnp.broadcast_to(xx[:, 1:2], shp)
    y_lo = jnp.broadcast_to(yy[:, 0:1], shp)
    y_hi = jnp.broadcast_to(yy[:, 1:2], shp)
    d_lo = jnp.broadcast_to(dd[:, 0:1], shp)
    d_hi = jnp.broadcast_to(dd[:, 1:2], shp)
    for j in range(1, NKNOT - 1):
        m = jnp.where(xx[:, j:j + 1] < x, 1.0, 0.0).astype(jnp.float32)
        x_lo = x_lo + m * dxx[:, j:j + 1]
        x_hi = x_hi + m * dxx[:, j + 1:j + 2]
        y_lo = y_lo + m * dyy[:, j:j + 1]
        y_hi = y_hi + m * dyy[:, j + 1:j + 2]
        d_lo = d_lo + m * ddl[:, j:j + 1]
        d_hi = d_hi + m * ddl[:, j + 1:j + 2]

    rdx = 1.0 / (x_hi - x_lo)
    xi = (x - x_lo) * rdx
    dy = y_hi - y_lo
    sl = dy * rdx
    xi1 = 1.0 - xi
    u = xi * xi1
    r = (sl + sl) * u
    den = (d_hi + d_lo) * u - r + sl
    rden = 1.0 / den
    xi2 = xi * xi
    y_mid = y_lo + dy * rden * (sl * xi2 + d_lo * u)
    num = d_lo * (xi1 * xi1) + (d_hi * xi2 + r)
    t = sl * rden
    ld_mid = jnp.log((t * t) * num)

    y_left = yy[:, 0:1] + (x - xx[:, 0:1]) * dd[:, 0:1]
    y_right = yy[:, -1:] + (x - xx[:, -1:]) * dd[:, -1:]
    sel0 = jnp.logical_not(xx[:, 0:1] < x)
    seln = xx[:, -1:] < x
    y_r[...] = jnp.where(sel0, y_left, jnp.where(seln, y_right, y_mid))
    ld_r[...] = jnp.where(
        sel0, jnp.broadcast_to(lds[:, 0:1], shp),
        jnp.where(seln, jnp.broadcast_to(lds[:, 1:2], shp), ld_mid))


def kernel(x, params):
    n = x.shape[0]
    assert x.shape == (n, NDIM) and params.shape == (3 * NDIM * NKNOT,)
    assert n == _NCOLS

    x0 = params[:NDIM]
    y0 = params[NDIM:2 * NDIM]
    off = 2 * NDIM
    logdx = params[off:off + NDIM * (NKNOT - 1)].reshape(NDIM, NKNOT - 1)
    off += NDIM * (NKNOT - 1)
    logdy = params[off:off + NDIM * (NKNOT - 1)].reshape(NDIM, NKNOT - 1)
    off += NDIM * (NKNOT - 1)
    logd = params[off:].reshape(NDIM, NKNOT)
    xx = jnp.concatenate(
        [x0[:, None], jnp.cumsum(jnp.exp(logdx), axis=1) + x0[:, None]], 1)
    yy = jnp.concatenate(
        [y0[:, None], jnp.cumsum(jnp.exp(logdy), axis=1) + y0[:, None]], 1)
    delta = jnp.exp(logd)
    cvals = jnp.stack(
        [xx[:, 0], xx[:, -1],
         delta[:, 0], yy[:, 0] - xx[:, 0] * delta[:, 0],
         delta[:, -1], yy[:, -1] - xx[:, -1] * delta[:, -1],
         logd[:, 0], logd[:, -1], xx[:, 15],
         jnp.zeros((NDIM,), jnp.float32)], axis=1)
    cpack = jnp.broadcast_to(
        cvals[:, :, None], (NDIM, _NC, LANES)).reshape(-1)
    tab = jnp.concatenate(
        [xx.reshape(-1), yy.reshape(-1), delta.reshape(-1), cpack], 0)

    mesh = plsc.VectorSubcoreMesh(core_axis_name="c", subcore_axis_name="s")
    out = jax.ShapeDtypeStruct((NDIM, n), jnp.float32)
    f = pl.kernel(
        _sc_body,
        out_type=[out, out],
        mesh=mesh,
        compiler_params=pltpu.CompilerParams(needs_layout_passes=False),
        scratch_types=[
            pltpu.VMEM((_T_LEN,), jnp.float32),
            pltpu.VMEM((2, NDIM, _CHUNK), jnp.float32),
            pltpu.VMEM((2, NDIM, _CHUNK), jnp.float32),
            pltpu.VMEM((2, NDIM, _CHUNK), jnp.float32),
            pltpu.SemaphoreType.DMA((2,)),
            pltpu.SemaphoreType.DMA((2,)),
            pltpu.SemaphoreType.DMA((2,)),
        ],
    )
    xt = x.T
    yt, ldt = f(xt, tab)

    dxx = jnp.concatenate(
        [jnp.zeros((NDIM, 1), jnp.float32), xx[:, 1:] - xx[:, :-1]], 1)
    dyy = jnp.concatenate(
        [jnp.zeros((NDIM, 1), jnp.float32), yy[:, 1:] - yy[:, :-1]], 1)
    ddl = jnp.concatenate(
        [jnp.zeros((NDIM, 1), jnp.float32), delta[:, 1:] - delta[:, :-1]], 1)
    lds = jnp.stack([logd[:, 0], logd[:, -1]], axis=1)
    nblocks = (_TC_COLS + _BC - 1) // _BC
    tab_spec = pl.BlockSpec((NDIM, NKNOT), lambda i: (0, 0))
    out_tc = jax.ShapeDtypeStruct((NDIM, _TC_COLS), jnp.float32)
    y_tc, ld_tc = pl.pallas_call(
        _tc_body,
        grid=(nblocks,),
        in_specs=[
            tab_spec, tab_spec, tab_spec,
            pl.BlockSpec((NDIM, 2), lambda i: (0, 0)),
            tab_spec, tab_spec, tab_spec,
            pl.BlockSpec((NDIM, _BC), lambda i: (0, _SC_COLS // _BC + i)),
        ],
        out_specs=[
            pl.BlockSpec((NDIM, _BC), lambda i: (0, i)),
            pl.BlockSpec((NDIM, _BC), lambda i: (0, i)),
        ],
        out_shape=[out_tc, out_tc],
    )(xx, yy, delta, lds, dxx, dyy, ddl, xt)

    yt = lax.dynamic_update_slice(yt, y_tc, (0, _SC_COLS))
    ldt = lax.dynamic_update_slice(ldt, ld_tc, (0, _SC_COLS))
    return yt.T, ldt.T


# speedup vs baseline: 2.2779x; 2.2779x over previous
import functools

import jax
import jax.numpy as jnp
from jax import lax
from jax.experimental import pallas as pl
from jax.experimental.pallas import tpu as pltpu
from jax.experimental.pallas import tpu_sc as plsc

NDIM = 16
NKNOT = 32
LANES = 16
NWORKERS = 32

_NC = 10
_T_XX = 0
_T_YY = NDIM * NKNOT
_T_D = 2 * NDIM * NKNOT
_T_C = 3 * NDIM * NKNOT
_T_LEN = 3 * NDIM * NKNOT + NDIM * _NC * LANES

_NCOLS = 1000000
_CHUNK = 512
_SC_K = 41
_SC_COLS = NWORKERS * _SC_K * _CHUNK
_TC_COLS = _NCOLS - _SC_COLS
_BC = 2048

_LN2 = 0.6931471805599453
_LOG_COEF = (
    9.09649109370747e-08,
    0.9999914499316742,
    -0.49980110932854144,
    0.33133371651864063,
    -0.2391898880613915,
    0.16478215592875825,
    -0.09231255283440946,
    0.03441802524375755,
    -0.006074773332369956,
)


def _softlog(a):
    i = lax.bitcast_convert_type(a, jnp.int32)
    e = (i >> 23) - 127
    z = lax.bitcast_convert_type(
        (i & 0x007FFFFF) | 0x3F800000, jnp.float32) - 1.0
    p = jnp.full((LANES,), _LOG_COEF[8], jnp.float32)
    for k in range(7, -1, -1):
        p = p * z + _LOG_COEF[k]
    return e.astype(jnp.float32) * _LN2 + p


def _spline_vec(tab_v, xv, bvec, b16, consts):
    xx0, xx31, d0, cleft, d31, cright, ld0, ld31, xmid = consts
    c = jnp.where(xmid < xv, b16, bvec)
    for s in (8, 4, 2, 1):
        v = plsc.load_gather(tab_v, [c + (s - 1)])
        c = jnp.where(v < xv, c + s, c)
    lo = jnp.maximum(c - 1, bvec)
    lo_y = lo + _T_YY
    lo_d = lo + _T_D
    x_lo = plsc.load_gather(tab_v, [lo])
    x_hi = plsc.load_gather(tab_v, [lo + 1])
    y_lo = plsc.load_gather(tab_v, [lo_y])
    y_hi = plsc.load_gather(tab_v, [lo_y + 1])
    d_lo = plsc.load_gather(tab_v, [lo_d])
    d_hi = plsc.load_gather(tab_v, [lo_d + 1])

    rdx = 1.0 / (x_hi - x_lo)
    xi = (xv - x_lo) * rdx
    dy = y_hi - y_lo
    sl = dy * rdx
    u = xi * (1.0 - xi)
    xi1 = 1.0 - xi
    r = (sl + sl) * u
    den = (d_hi + d_lo) * u - r + sl
    rden = 1.0 / den
    xi2 = xi * xi
    y_mid = y_lo + dy * rden * (sl * xi2 + d_lo * u)
    num = d_lo * (xi1 * xi1) + (d_hi * xi2 + r)
    t = sl * rden
    ld_mid = _softlog((t * t) * num)

    y_left = cleft + xv * d0
    y_right = cright + xv * d31
    sel0 = jnp.logical_not(xx0 < xv)
    seln = xx31 < xv
    y = jnp.where(sel0, y_left, jnp.where(seln, y_right, y_mid))
    ld = jnp.where(sel0, ld0, jnp.where(seln, ld31, ld_mid))
    return y, ld


def _do_chunk(tab_v, xb, yb, ldb, ncols):
    per_dim = ncols // LANES
    shift = per_dim.bit_length() - 1
    assert per_dim == 1 << shift

    @plsc.parallel_loop(0, NDIM * per_dim, unroll=16)
    def vec_body(j):
        d = j >> shift
        sl_ = pl.ds((j - (d << shift)) * LANES, LANES)
        cbase = _T_C + d * (_NC * LANES)
        consts = tuple(
            tab_v[pl.ds(cbase + i * LANES, LANES)] for i in range(9))
        bvec = jnp.full((LANES,), d * NKNOT, jnp.int32)
        b16 = jnp.full((LANES,), d * NKNOT + 16, jnp.int32)
        y, ld = _spline_vec(tab_v, xb[d, sl_], bvec, b16, consts)
        yb[d, sl_] = y
        ldb[d, sl_] = ld


def _sc_body(x_hbm, tab_hbm, y_hbm, ld_hbm,
             tab_v, xb, yb, ldb, sin, soy, sold):
    wid = lax.axis_index("s") * 2 + lax.axis_index("c")
    base = wid * (_SC_K * _CHUNK)
    nchunks = _SC_K

    pltpu.sync_copy(tab_hbm, tab_v)

    def start_in(g):
        b = g & 1
        c0 = base + g * _CHUNK
        pltpu.async_copy(x_hbm.at[:, pl.ds(c0, _CHUNK)], xb.at[b],
                         sin.at[b])

    start_in(0)
    start_in(1)

    def chunk_body(g, carry):
        b = g & 1
        c0 = base + g * _CHUNK
        pltpu.make_async_copy(x_hbm.at[:, pl.ds(c0, _CHUNK)], xb.at[b],
                              sin.at[b]).wait()

        @pl.when(g >= 2)
        def _drain():
            c0p = c0 - 2 * _CHUNK
            pltpu.make_async_copy(yb.at[b], y_hbm.at[:, pl.ds(c0p, _CHUNK)],
                                  soy.at[b]).wait()
            pltpu.make_async_copy(ldb.at[b], ld_hbm.at[:, pl.ds(c0p, _CHUNK)],
                                  sold.at[b]).wait()

        _do_chunk(tab_v, xb.at[b], yb.at[b], ldb.at[b], _CHUNK)
        pltpu.async_copy(yb.at[b], y_hbm.at[:, pl.ds(c0, _CHUNK)], soy.at[b])
        pltpu.async_copy(ldb.at[b], ld_hbm.at[:, pl.ds(c0, _CHUNK)],
                         sold.at[b])

        @pl.when(g + 2 < nchunks)
        def _prefetch():
            start_in(g + 2)

        return carry

    lax.fori_loop(0, nchunks, chunk_body, 0)

    for off in (2, 1):
        g = nchunks - off
        b = g & 1
        c0 = base + g * _CHUNK
        pltpu.make_async_copy(yb.at[b], y_hbm.at[:, pl.ds(c0, _CHUNK)],
                              soy.at[b]).wait()
        pltpu.make_async_copy(ldb.at[b], ld_hbm.at[:, pl.ds(c0, _CHUNK)],
                              sold.at[b]).wait()


def _tc_body(xx_r, yy_r, dd_r, lds_r, dxx_r, dyy_r, ddl_r,
             x_r, y_r, ld_r):
    x = x_r[...]
    xx = xx_r[...]
    yy = yy_r[...]
    dd = dd_r[...]
    lds = lds_r[...]
    dxx = dxx_r[...]
    dyy = dyy_r[...]
    ddl = ddl_r[...]
    shp = x.shape

    x_lo = jnp.broadcast_to(xx[:, 0:1], shp)
    x_hi = jnp.broadcast_to(xx[:, 1:2], shp)
    y_lo = jnp.broadcast_to(yy[:, 0:1], shp)
    y_hi = jnp.broadcast_to(yy[:, 1:2], shp)
    d_lo = jnp.broadcast_to(dd[:, 0:1], shp)
    d_hi = jnp.broadcast_to(dd[:, 1:2], shp)
    for j in range(1, NKNOT - 1):
        m = jnp.where(xx[:, j:j + 1] < x, 1.0, 0.0).astype(jnp.float32)
        x_lo = x_lo + m * dxx[:, j:j + 1]
        x_hi = x_hi + m * dxx[:, j + 1:j + 2]
        y_lo = y_lo + m * dyy[:, j:j + 1]
        y_hi = y_hi + m * dyy[:, j + 1:j + 2]
        d_lo = d_lo + m * ddl[:, j:j + 1]
        d_hi = d_hi + m * ddl[:, j + 1:j + 2]

    rdx = 1.0 / (x_hi - x_lo)
    xi = (x - x_lo) * rdx
    dy = y_hi - y_lo
    sl = dy * rdx
    xi1 = 1.0 - xi
    u = xi * xi1
    r = (sl + sl) * u
    den = (d_hi + d_lo) * u - r + sl
    rden = 1.0 / den
    xi2 = xi * xi
    y_mid = y_lo + dy * rden * (sl * xi2 + d_lo * u)
    num = d_lo * (xi1 * xi1) + (d_hi * xi2 + r)
    t = sl * rden
    ld_mid = jnp.log((t * t) * num)

    y_left = yy[:, 0:1] + (x - xx[:, 0:1]) * dd[:, 0:1]
    y_right = yy[:, -1:] + (x - xx[:, -1:]) * dd[:, -1:]
    sel0 = jnp.logical_not(xx[:, 0:1] < x)
    seln = xx[:, -1:] < x
    y_r[...] = jnp.where(sel0, y_left, jnp.where(seln, y_right, y_mid))
    ld_r[...] = jnp.where(
        sel0, jnp.broadcast_to(lds[:, 0:1], shp),
        jnp.where(seln, jnp.broadcast_to(lds[:, 1:2], shp), ld_mid))


def kernel(x, params):
    n = x.shape[0]
    assert x.shape == (n, NDIM) and params.shape == (3 * NDIM * NKNOT,)
    assert n == _NCOLS

    x0 = params[:NDIM]
    y0 = params[NDIM:2 * NDIM]
    off = 2 * NDIM
    logdx = params[off:off + NDIM * (NKNOT - 1)].reshape(NDIM, NKNOT - 1)
    off += NDIM * (NKNOT - 1)
    logdy = params[off:off + NDIM * (NKNOT - 1)].reshape(NDIM, NKNOT - 1)
    off += NDIM * (NKNOT - 1)
    logd = params[off:].reshape(NDIM, NKNOT)
    xx = jnp.concatenate(
        [x0[:, None], jnp.cumsum(jnp.exp(logdx), axis=1) + x0[:, None]], 1)
    yy = jnp.concatenate(
        [y0[:, None], jnp.cumsum(jnp.exp(logdy), axis=1) + y0[:, None]], 1)
    delta = jnp.exp(logd)
    cvals = jnp.stack(
        [xx[:, 0], xx[:, -1],
         delta[:, 0], yy[:, 0] - xx[:, 0] * delta[:, 0],
         delta[:, -1], yy[:, -1] - xx[:, -1] * delta[:, -1],
         logd[:, 0], logd[:, -1], xx[:, 15],
         jnp.zeros((NDIM,), jnp.float32)], axis=1)
    cpack = jnp.broadcast_to(
        cvals[:, :, None], (NDIM, _NC, LANES)).reshape(-1)
    tab = jnp.concatenate(
        [xx.reshape(-1), yy.reshape(-1), delta.reshape(-1), cpack], 0)

    mesh = plsc.VectorSubcoreMesh(core_axis_name="c", subcore_axis_name="s")
    out = jax.ShapeDtypeStruct((NDIM, n), jnp.float32)
    f = pl.kernel(
        _sc_body,
        out_type=[out, out],
        mesh=mesh,
        compiler_params=pltpu.CompilerParams(needs_layout_passes=False),
        scratch_types=[
            pltpu.VMEM((_T_LEN,), jnp.float32),
            pltpu.VMEM((2, NDIM, _CHUNK), jnp.float32),
            pltpu.VMEM((2, NDIM, _CHUNK), jnp.float32),
            pltpu.VMEM((2, NDIM, _CHUNK), jnp.float32),
            pltpu.SemaphoreType.DMA((2,)),
            pltpu.SemaphoreType.DMA((2,)),
            pltpu.SemaphoreType.DMA((2,)),
        ],
    )
    xt = x.T
    yt, ldt = f(xt, tab)

    dxx = jnp.concatenate(
        [jnp.zeros((NDIM, 1), jnp.float32), xx[:, 1:] - xx[:, :-1]], 1)
    dyy = jnp.concatenate(
        [jnp.zeros((NDIM, 1), jnp.float32), yy[:, 1:] - yy[:, :-1]], 1)
    ddl = jnp.concatenate(
        [jnp.zeros((NDIM, 1), jnp.float32), delta[:, 1:] - delta[:, :-1]], 1)
    lds = jnp.stack([logd[:, 0], logd[:, -1]], axis=1)
    nblocks = (_TC_COLS + _BC - 1) // _BC
    tab_spec = pl.BlockSpec((NDIM, NKNOT), lambda i: (0, 0))
    out_tc = jax.ShapeDtypeStruct((NDIM, _TC_COLS), jnp.float32)
    y_tc, ld_tc = pl.pallas_call(
        _tc_body,
        grid=(nblocks,),
        in_specs=[
            tab_spec, tab_spec, tab_spec,
            pl.BlockSpec((NDIM, 2), lambda i: (0, 0)),
            tab_spec, tab_spec, tab_spec,
            pl.BlockSpec((NDIM, _BC), lambda i: (0, _SC_COLS // _BC + i)),
        ],
        out_specs=[
            pl.BlockSpec((NDIM, _BC), lambda i: (0, i)),
            pl.BlockSpec((NDIM, _BC), lambda i: (0, i)),
        ],
        out_shape=[out_tc, out_tc],
    )(xx, yy, delta, lds, dxx, dyy, ddl, xt)

    yt = lax.dynamic_update_slice(yt, y_tc, (0, _SC_COLS))
    ldt = lax.dynamic_update_slice(ldt, ld_tc, (0, _SC_COLS))
    return yt.T, ldt.T
